# baseline (speedup 1.0000x reference)
import functools
import math

import jax
import jax.numpy as jnp
from jax import lax
from jax.experimental import pallas as pl
from jax.experimental.pallas import tpu as pltpu
from jax.experimental.pallas import tpu_sc as plsc

N = 10000
E = 320000
D = 128
A = 8
EE = 16
EA = 4
HID = 8

_NC = 2
_NS = 16
_NW = _NC * _NS
_CH = 64
_NCHUNK = E // _CH
_CBASE = _NCHUNK // _NW
_CEXTRA = _NCHUNK - _CBASE * _NW
_RPT = 624


def _pack_cols(y):
    lo = lax.bitcast_convert_type(y[:, :D // 2].astype(jnp.bfloat16),
                                  jnp.uint16).astype(jnp.uint32)
    hi = lax.bitcast_convert_type(y[:, D // 2:].astype(jnp.bfloat16),
                                  jnp.uint16).astype(jnp.uint32)
    return lax.bitcast_convert_type(lo | (hi << 16), jnp.int32)


def _nf_body(x_ref, w_ref, o_ref):
    o_ref[...] = (x_ref[...] @ w_ref[...]) * (1.0 / math.sqrt(D))


def _wsum_body(eeT_ref, eaT_ref, w0_ref, wg_ref, r1_ref, r2_ref, o_ref):
    dn0 = (((0,), (0,)), ((), ()))
    f32 = jnp.float32
    hT = jax.nn.silu(
        lax.dot_general(w0_ref[...], eeT_ref[...], dn0,
                        preferred_element_type=f32) * (1.0 / math.sqrt(EE)))
    g1 = lax.dot_general(r1_ref[...], hT, dn0, preferred_element_type=f32)
    g2 = lax.dot_general(r2_ref[...], eaT_ref[...], dn0,
                         preferred_element_type=f32)
    o_ref[...] = _pack_cols(
        lax.dot_general(g1 * g2, wg_ref[...], dn0,
                        preferred_element_type=f32) * (1.0 / 32.0))


def _out_body(pa_ref, pb_ref, nf_ref, na_ref, w2_ref, wsc_ref, o_ref):
    acc = ((pa_ref[...] + pb_ref[...]) @ w2_ref[...]) * (1.0 / math.sqrt(D))
    nf = nf_ref[...]
    na = na_ref[...]
    s = (nf * na[:, 0:1]) @ wsc_ref[0]
    for j in range(1, A):
        s = s + (nf * na[:, j:j + 1]) @ wsc_ref[j]
    o_ref[...] = acc + s * (1.0 / math.sqrt(float(D * A)))


def _sc_body(nf_hbm, wsum_hbm, ei_hbm, out_hbm,
             idx_v, xe_v, m_v, p_v, acc_sh, *sems):
    sg = sems[0:2]
    sw = sems[2:4]
    ss = sems[4:6]
    c = lax.axis_index("c")
    s = lax.axis_index("s")
    w = c * _NS + s

    def _zrow(r, carry):
        for q in range(D // 16):
            p_v[0, r, pl.ds(q * 16, 16)] = jnp.zeros((16,), jnp.float32)
        return carry
    lax.fori_loop(0, _CH, _zrow, 0)
    for k in range(_RPT // _CH):
        pltpu.sync_copy(p_v.at[0, pl.ds(0, _CH)],
                        acc_sh.at[pl.ds(s * _RPT + k * _CH, _CH)])
    _zfull = (_RPT // _CH) * _CH
    _zrem = _RPT - _zfull

    @pl.when(s < _NS - 1)
    def _():
        pltpu.sync_copy(p_v.at[0, pl.ds(0, _zrem)],
                        acc_sh.at[pl.ds(s * _RPT + _zfull, _zrem)])

    @pl.when(s == _NS - 1)
    def _():
        pltpu.sync_copy(p_v.at[0, pl.ds(0, _CH)],
                        acc_sh.at[pl.ds(s * _RPT + _zfull, _CH)])
    plsc.subcore_barrier()

    start = w * _CBASE

    def _issue(b, i):
        base = pl.multiple_of((start + i) * _CH, _CH)
        pltpu.sync_copy(ei_hbm.at[pl.ds(base, _CH)], idx_v.at[2 * b])
        pltpu.sync_copy(ei_hbm.at[pl.ds(E + base, _CH)], idx_v.at[2 * b + 1])
        pltpu.async_copy(nf_hbm.at[idx_v.at[2 * b]], xe_v.at[b], sg[b])
        pltpu.async_copy(wsum_hbm.at[pl.ds(base, _CH)], m_v.at[b], sw[b])

    def _multiply(b):
        mask = jnp.full((16,), -65536, jnp.int32)

        def _mrow(r, cc):
            for q in range(D // 32):
                sl = pl.ds(q * 16, 16)
                mi = m_v[b, r, sl]
                ml = plsc.bitcast(mi << 16, jnp.float32)
                mh = plsc.bitcast(mi & mask, jnp.float32)
                xl = xe_v[b, r, pl.ds(q * 16, 16)]
                xh = xe_v[b, r, pl.ds(D // 2 + q * 16, 16)]
                p_v[b, r, pl.ds(q * 16, 16)] = ml * xl
                p_v[b, r, pl.ds(D // 2 + q * 16, 16)] = mh * xh
            return cc
        lax.fori_loop(0, _CH, _mrow, 0)

    _issue(0, 0)

    @pl.loop(0, _CBASE, step=2)
    def _pair(k):
        for b in (0, 1):
            i = k + b
            pltpu.make_async_copy(nf_hbm.at[pl.ds(0, _CH)],
                                  xe_v.at[b], sg[b]).wait()
            pltpu.make_async_copy(wsum_hbm.at[pl.ds(0, _CH)],
                                  m_v.at[b], sw[b]).wait()

            @pl.when(i + 1 < _CBASE)
            def _():
                _issue(1 - b, i + 1)

            @pl.when(i >= 2)
            def _():
                pltpu.make_async_copy(out_hbm.at[0, pl.ds(0, _CH)],
                                      p_v.at[b], ss[b]).wait()
            _multiply(b)
            for q in range(_CH // 16):
                sl = pl.ds(q * 16, 16)
                idx_v[4 + b, sl] = idx_v[2 * b + 1, sl]
            pltpu.async_copy(p_v.at[b], acc_sh.at[idx_v.at[4 + b]], ss[b],
                             add=True)

    for b in (0, 1):
        pltpu.make_async_copy(out_hbm.at[0, pl.ds(0, _CH)],
                              p_v.at[b], ss[b]).wait()

    @pl.when(w < _CEXTRA)
    def _():
        base = pl.multiple_of((_NW * _CBASE + w) * _CH, _CH)
        pltpu.sync_copy(ei_hbm.at[pl.ds(base, _CH)], idx_v.at[0])
        pltpu.sync_copy(ei_hbm.at[pl.ds(E + base, _CH)], idx_v.at[1])
        pltpu.async_copy(nf_hbm.at[idx_v.at[0]], xe_v.at[0], sg[0]).wait()
        pltpu.sync_copy(wsum_hbm.at[pl.ds(base, _CH)], m_v.at[0])
        _multiply(0)
        pltpu.sync_copy(p_v.at[0], acc_sh.at[idx_v.at[1]], add=True)

    plsc.subcore_barrier()

    @pl.when(s < _NS - 1)
    def _():
        pltpu.sync_copy(acc_sh.at[pl.ds(s * _RPT, _RPT)],
                        out_hbm.at[c, pl.ds(s * _RPT, _RPT)])

    @pl.when(s == _NS - 1)
    def _():
        pltpu.sync_copy(acc_sh.at[pl.ds(s * _RPT, N - (_NS - 1) * _RPT)],
                        out_hbm.at[c, pl.ds(s * _RPT, N - (_NS - 1) * _RPT)])


@functools.lru_cache(maxsize=1)
def _sc_gather_scatter():
    return functools.partial(
        pl.kernel,
        out_type=jax.ShapeDtypeStruct((_NC, N, D), jnp.float32),
        mesh=plsc.VectorSubcoreMesh(core_axis_name="c", subcore_axis_name="s",
                                    num_cores=_NC, num_subcores=_NS),
        compiler_params=pltpu.CompilerParams(needs_layout_passes=False),
        scratch_types=[
            pltpu.VMEM((6, _CH), jnp.int32),
            pltpu.VMEM((2, _CH, D), jnp.float32),
            pltpu.VMEM((2, _CH, D // 2), jnp.int32),
            pltpu.VMEM((2, _CH, D), jnp.float32),
            pltpu.VMEM_SHARED((N, D), jnp.float32),
            pltpu.SemaphoreType.DMA,
            pltpu.SemaphoreType.DMA,
            pltpu.SemaphoreType.DMA,
            pltpu.SemaphoreType.DMA,
            pltpu.SemaphoreType.DMA,
            pltpu.SemaphoreType.DMA,
        ],
    )(_sc_body)


def kernel(node_features, node_attrs, edge_embedding, edge_attrs, edge_index,
           W1, mlp_w0, mlp_w1, W2, W_sc):
    f32 = jnp.float32
    ei_flat = edge_index.astype(jnp.int32).reshape(2 * E)

    Wg = mlp_w1.reshape(HID, D, EA).transpose(0, 2, 1).reshape(HID * EA, D)
    Wsc = W_sc.transpose(1, 0, 2)
    R1 = (jnp.arange(HID)[:, None]
          == (jnp.arange(HID * EA)[None, :] // EA)).astype(f32)
    R2 = (jnp.arange(EA)[:, None]
          == (jnp.arange(HID * EA)[None, :] % EA)).astype(f32)

    TN = 2000
    node_feats = pl.pallas_call(
        _nf_body,
        grid=(N // TN,),
        in_specs=[pl.BlockSpec((TN, D), lambda i: (i, 0)),
                  pl.BlockSpec((D, D), lambda i: (0, 0))],
        out_specs=pl.BlockSpec((TN, D), lambda i: (i, 0)),
        out_shape=jax.ShapeDtypeStruct((N, D), f32),
    )(node_features, W1)

    TE = 12800
    wsum = pl.pallas_call(
        _wsum_body,
        grid=(E // TE,),
        in_specs=[pl.BlockSpec((EE, TE), lambda i: (0, i)),
                  pl.BlockSpec((EA, TE), lambda i: (0, i)),
                  pl.BlockSpec((EE, HID), lambda i: (0, 0)),
                  pl.BlockSpec((HID * EA, D), lambda i: (0, 0)),
                  pl.BlockSpec((HID, HID * EA), lambda i: (0, 0)),
                  pl.BlockSpec((EA, HID * EA), lambda i: (0, 0))],
        out_specs=pl.BlockSpec((TE, D // 2), lambda i: (i, 0)),
        out_shape=jax.ShapeDtypeStruct((E, D // 2), jnp.int32),
    )(edge_embedding.T, edge_attrs.T, mlp_w0, Wg, R1, R2)

    partials = _sc_gather_scatter()(node_feats, wsum, ei_flat)

    out = pl.pallas_call(
        _out_body,
        grid=(N // TN,),
        in_specs=[pl.BlockSpec((TN, D), lambda i: (i, 0)),
                  pl.BlockSpec((TN, D), lambda i: (i, 0)),
                  pl.BlockSpec((TN, D), lambda i: (i, 0)),
                  pl.BlockSpec((TN, A), lambda i: (i, 0)),
                  pl.BlockSpec((D, D), lambda i: (0, 0)),
                  pl.BlockSpec((A, D, D), lambda i: (0, 0, 0))],
        out_specs=pl.BlockSpec((TN, D), lambda i: (i, 0)),
        out_shape=jax.ShapeDtypeStruct((N, D), f32),
    )(partials[0], partials[1], node_features, node_attrs, W2, Wsc)
    return out


# speedup vs baseline: 1.3591x; 1.3591x over previous
import functools
import math

import jax
import jax.numpy as jnp
from jax import lax
from jax.experimental import pallas as pl
from jax.experimental.pallas import tpu as pltpu
from jax.experimental.pallas import tpu_sc as plsc

N = 10000
E = 320000
D = 128
A = 8
EE = 16
EA = 4
HID = 8

_NC = 2
_NS = 16
_NW = _NC * _NS
_CH = 64
_NCHUNK = E // _CH
_CBASE = _NCHUNK // _NW
_CEXTRA = _NCHUNK - _CBASE * _NW
_S = 3
_NSUP = _CBASE // _S
_RPT = 624


def _pack_cols(y):
    lo = lax.bitcast_convert_type(y[:, :D // 2].astype(jnp.bfloat16),
                                  jnp.uint16).astype(jnp.uint32)
    hi = lax.bitcast_convert_type(y[:, D // 2:].astype(jnp.bfloat16),
                                  jnp.uint16).astype(jnp.uint32)
    return lax.bitcast_convert_type(lo | (hi << 16), jnp.int32)


def _nf_body(x_ref, w_ref, o_ref):
    o_ref[...] = (x_ref[...] @ w_ref[...]) * (1.0 / math.sqrt(D))


def _wsum_body(eeT_ref, eaT_ref, w0_ref, wg_ref, r1_ref, r2_ref, o_ref):
    dn0 = (((0,), (0,)), ((), ()))
    f32 = jnp.float32
    hT = jax.nn.silu(
        lax.dot_general(w0_ref[...], eeT_ref[...], dn0,
                        preferred_element_type=f32) * (1.0 / math.sqrt(EE)))
    g1 = lax.dot_general(r1_ref[...], hT, dn0, preferred_element_type=f32)
    g2 = lax.dot_general(r2_ref[...], eaT_ref[...], dn0,
                         preferred_element_type=f32)
    o_ref[...] = _pack_cols(
        lax.dot_general(g1 * g2, wg_ref[...], dn0,
                        preferred_element_type=f32) * (1.0 / 32.0))


def _out_body(pa_ref, pb_ref, nf_ref, na_ref, w2_ref, wsc_ref, o_ref):
    acc = ((pa_ref[...] + pb_ref[...]) @ w2_ref[...]) * (1.0 / math.sqrt(D))
    nf = nf_ref[...]
    na = na_ref[...]
    s = (nf * na[:, 0:1]) @ wsc_ref[0]
    for j in range(1, A):
        s = s + (nf * na[:, j:j + 1]) @ wsc_ref[j]
    o_ref[...] = acc + s * (1.0 / math.sqrt(float(D * A)))


def _sc_body(nf_hbm, wsum_hbm, ei_hbm, out_hbm,
             sidx0_v, sidx1_v, didx0_v, didx1_v, pdst_v, psrc_v, xe_v, m_v,
             p_v, acc_sh, *sems):
    sidx = (sidx0_v, sidx1_v)
    didx = (didx0_v, didx1_v)
    sg = sems[0:2]
    sw = sems[2:4]
    ss = sems[4:6]
    si = sems[6:8]
    c = lax.axis_index("c")
    s = lax.axis_index("s")
    w = c * _NS + s

    def _zrow(r, carry):
        for q in range(D // 16):
            p_v[0, r, pl.ds(q * 16, 16)] = jnp.zeros((16,), jnp.float32)
        return carry
    lax.fori_loop(0, _CH, _zrow, 0)
    for k in range(_RPT // _CH):
        pltpu.sync_copy(p_v.at[0, pl.ds(0, _CH)],
                        acc_sh.at[pl.ds(s * _RPT + k * _CH, _CH)])
    _zfull = (_RPT // _CH) * _CH
    _zrem = _RPT - _zfull

    @pl.when(s < _NS - 1)
    def _():
        pltpu.sync_copy(p_v.at[0, pl.ds(0, _zrem)],
                        acc_sh.at[pl.ds(s * _RPT + _zfull, _zrem)])

    @pl.when(s == _NS - 1)
    def _():
        pltpu.sync_copy(p_v.at[0, pl.ds(0, _CH)],
                        acc_sh.at[pl.ds(s * _RPT + _zfull, _CH)])
    plsc.subcore_barrier()

    start = w * _CBASE
    _SB = _S * _CH

    def _load_super(bb, t, sem):
        base = pl.multiple_of((start + t * _S) * _CH, _CH)
        if sem is None:
            pltpu.sync_copy(ei_hbm.at[pl.ds(base, _SB)], sidx[bb])
            pltpu.sync_copy(ei_hbm.at[pl.ds(E + base, _SB)], didx[bb])
        else:
            pltpu.async_copy(ei_hbm.at[pl.ds(base, _SB)], sidx[bb], sem)
            pltpu.async_copy(ei_hbm.at[pl.ds(E + base, _SB)], didx[bb], sem)

    def _stash_src(bp, bb, jloc):
        for q in range(_CH // 16):
            psrc_v[bp, pl.ds(q * 16, 16)] = (
                sidx[bb][pl.ds(jloc * _CH + q * 16, 16)])

    def _issue(b, i):
        base = pl.multiple_of((start + i) * _CH, _CH)
        pltpu.async_copy(nf_hbm.at[psrc_v.at[b]], xe_v.at[b], sg[b])
        pltpu.async_copy(wsum_hbm.at[pl.ds(base, _CH)], m_v.at[b], sw[b])

    def _multiply(b):
        mask = jnp.full((16,), -65536, jnp.int32)

        def _mrow(r, cc):
            for q in range(D // 32):
                sl = pl.ds(q * 16, 16)
                mi = m_v[b, r, sl]
                ml = plsc.bitcast(mi << 16, jnp.float32)
                mh = plsc.bitcast(mi & mask, jnp.float32)
                xl = xe_v[b, r, pl.ds(q * 16, 16)]
                xh = xe_v[b, r, pl.ds(D // 2 + q * 16, 16)]
                p_v[b, r, pl.ds(q * 16, 16)] = ml * xl
                p_v[b, r, pl.ds(D // 2 + q * 16, 16)] = mh * xh
            return cc
        lax.fori_loop(0, _CH, _mrow, 0)

    _load_super(0, 0, None)
    _stash_src(0, 0, 0)
    _issue(0, 0)

    @pl.loop(0, _NSUP, step=2)
    def _superpair(k):
        for dt in (0, 1):
            t = k + dt
            bb = dt
            for j in range(_S):
                b = (dt + j) % 2
                i = t * _S + j
                pltpu.make_async_copy(nf_hbm.at[pl.ds(0, _CH)],
                                      xe_v.at[b], sg[b]).wait()
                pltpu.make_async_copy(wsum_hbm.at[pl.ds(0, _CH)],
                                      m_v.at[b], sw[b]).wait()
                if j == 0:
                    if dt == 0:
                        _load_super(1 - bb, t + 1, si[1 - bb])
                    else:
                        @pl.when(k < _NSUP - 2)
                        def _():
                            _load_super(1 - bb, t + 1, si[1 - bb])
                if j < _S - 1:
                    _stash_src(1 - b, bb, j + 1)
                    _issue(1 - b, i + 1)
                else:
                    def _nxt():
                        pltpu.make_async_copy(ei_hbm.at[pl.ds(0, _SB)],
                                              sidx[1 - bb], si[1 - bb]).wait()
                        pltpu.make_async_copy(ei_hbm.at[pl.ds(0, _SB)],
                                              didx[1 - bb], si[1 - bb]).wait()
                        _stash_src(1 - b, 1 - bb, 0)
                        _issue(1 - b, i + 1)
                    if dt == 0:
                        _nxt()
                    else:
                        pl.when(k < _NSUP - 2)(_nxt)

                @pl.when(i >= 2)
                def _():
                    pltpu.make_async_copy(out_hbm.at[0, pl.ds(0, _CH)],
                                          p_v.at[b], ss[b]).wait()
                _multiply(b)
                for q in range(_CH // 16):
                    sl = pl.ds(q * 16, 16)
                    pdst_v[b, sl] = didx[bb][pl.ds(j * _CH + q * 16, 16)]
                pltpu.async_copy(p_v.at[b], acc_sh.at[pdst_v.at[b]], ss[b],
                                 add=True)

    for b in (0, 1):
        pltpu.make_async_copy(out_hbm.at[0, pl.ds(0, _CH)],
                              p_v.at[b], ss[b]).wait()

    @pl.when(w < _CEXTRA)
    def _():
        base = pl.multiple_of((_NW * _CBASE + w) * _CH, _CH)
        pltpu.sync_copy(ei_hbm.at[pl.ds(base, _CH)], psrc_v.at[0])
        pltpu.sync_copy(ei_hbm.at[pl.ds(E + base, _CH)], pdst_v.at[0])
        pltpu.async_copy(nf_hbm.at[psrc_v.at[0]],
                         xe_v.at[0], sg[0]).wait()
        pltpu.sync_copy(wsum_hbm.at[pl.ds(base, _CH)], m_v.at[0])
        _multiply(0)
        pltpu.sync_copy(p_v.at[0], acc_sh.at[pdst_v.at[0]], add=True)

    plsc.subcore_barrier()

    @pl.when(s < _NS - 1)
    def _():
        pltpu.sync_copy(acc_sh.at[pl.ds(s * _RPT, _RPT)],
                        out_hbm.at[c, pl.ds(s * _RPT, _RPT)])

    @pl.when(s == _NS - 1)
    def _():
        pltpu.sync_copy(acc_sh.at[pl.ds(s * _RPT, N - (_NS - 1) * _RPT)],
                        out_hbm.at[c, pl.ds(s * _RPT, N - (_NS - 1) * _RPT)])


@functools.lru_cache(maxsize=1)
def _sc_gather_scatter():
    return functools.partial(
        pl.kernel,
        out_type=jax.ShapeDtypeStruct((_NC, N, D), jnp.float32),
        mesh=plsc.VectorSubcoreMesh(core_axis_name="c", subcore_axis_name="s",
                                    num_cores=_NC, num_subcores=_NS),
        compiler_params=pltpu.CompilerParams(needs_layout_passes=False),
        scratch_types=[
            pltpu.VMEM((_S * _CH,), jnp.int32),
            pltpu.VMEM((_S * _CH,), jnp.int32),
            pltpu.VMEM((_S * _CH,), jnp.int32),
            pltpu.VMEM((_S * _CH,), jnp.int32),
            pltpu.VMEM((2, _CH), jnp.int32),
            pltpu.VMEM((2, _CH), jnp.int32),
            pltpu.VMEM((2, _CH, D), jnp.float32),
            pltpu.VMEM((2, _CH, D // 2), jnp.int32),
            pltpu.VMEM((2, _CH, D), jnp.float32),
            pltpu.VMEM_SHARED((N, D), jnp.float32),
            pltpu.SemaphoreType.DMA,
            pltpu.SemaphoreType.DMA,
            pltpu.SemaphoreType.DMA,
            pltpu.SemaphoreType.DMA,
            pltpu.SemaphoreType.DMA,
            pltpu.SemaphoreType.DMA,
            pltpu.SemaphoreType.DMA,
            pltpu.SemaphoreType.DMA,
        ],
    )(_sc_body)


def kernel(node_features, node_attrs, edge_embedding, edge_attrs, edge_index,
           W1, mlp_w0, mlp_w1, W2, W_sc):
    f32 = jnp.float32
    ei_flat = edge_index.astype(jnp.int32).reshape(2 * E)

    Wg = mlp_w1.reshape(HID, D, EA).transpose(0, 2, 1).reshape(HID * EA, D)
    Wsc = W_sc.transpose(1, 0, 2)
    R1 = (jnp.arange(HID)[:, None]
          == (jnp.arange(HID * EA)[None, :] // EA)).astype(f32)
    R2 = (jnp.arange(EA)[:, None]
          == (jnp.arange(HID * EA)[None, :] % EA)).astype(f32)

    TN = 2000
    node_feats = pl.pallas_call(
        _nf_body,
        grid=(N // TN,),
        in_specs=[pl.BlockSpec((TN, D), lambda i: (i, 0)),
                  pl.BlockSpec((D, D), lambda i: (0, 0))],
        out_specs=pl.BlockSpec((TN, D), lambda i: (i, 0)),
        out_shape=jax.ShapeDtypeStruct((N, D), f32),
    )(node_features, W1)

    TE = 12800
    wsum = pl.pallas_call(
        _wsum_body,
        grid=(E // TE,),
        in_specs=[pl.BlockSpec((EE, TE), lambda i: (0, i)),
                  pl.BlockSpec((EA, TE), lambda i: (0, i)),
                  pl.BlockSpec((EE, HID), lambda i: (0, 0)),
                  pl.BlockSpec((HID * EA, D), lambda i: (0, 0)),
                  pl.BlockSpec((HID, HID * EA), lambda i: (0, 0)),
                  pl.BlockSpec((EA, HID * EA), lambda i: (0, 0))],
        out_specs=pl.BlockSpec((TE, D // 2), lambda i: (i, 0)),
        out_shape=jax.ShapeDtypeStruct((E, D // 2), jnp.int32),
    )(edge_embedding.T, edge_attrs.T, mlp_w0, Wg, R1, R2)

    partials = _sc_gather_scatter()(node_feats, wsum, ei_flat)

    out = pl.pallas_call(
        _out_body,
        grid=(N // TN,),
        in_specs=[pl.BlockSpec((TN, D), lambda i: (i, 0)),
                  pl.BlockSpec((TN, D), lambda i: (i, 0)),
                  pl.BlockSpec((TN, D), lambda i: (i, 0)),
                  pl.BlockSpec((TN, A), lambda i: (i, 0)),
                  pl.BlockSpec((D, D), lambda i: (0, 0)),
                  pl.BlockSpec((A, D, D), lambda i: (0, 0, 0))],
        out_specs=pl.BlockSpec((TN, D), lambda i: (i, 0)),
        out_shape=jax.ShapeDtypeStruct((N, D), f32),
    )(partials[0], partials[1], node_features, node_attrs, W2, Wsc)
    return out
